# baseline (speedup 1.0000x reference)
import jax
import jax.numpy as jnp
from jax import lax
from jax.experimental import pallas as pl
from jax.experimental.pallas import tpu as pltpu
from jax.experimental.pallas import tpu_sc as plsc

NUM_E = 64
TOPK = 8
LANES = 16
NC = 2
NS = 16
NW = NC * NS


def _fold_sum(vals):
    vals = list(vals)
    p = 1
    while p < len(vals):
        p *= 2
    while p > 1:
        p //= 2
        nxt = []
        for i in range(p):
            if i + p < len(vals):
                nxt.append(vals[i] + vals[i + p])
            elif i < len(vals):
                nxt.append(vals[i])
        vals = nxt
    return vals[0]


def _sc_body(conf_hbm, wealth_hbm, sel_hbm, rw_hbm, pay_hbm,
             conf_v, wealth_v, bids_v, sel_v, rw_v, pay_v):
    tpw = conf_v.shape[0] // NUM_E
    wid = lax.axis_index("s") * NC + lax.axis_index("c")
    base = wid * tpw
    pltpu.sync_copy(conf_hbm.at[pl.ds(base * NUM_E, tpw * NUM_E)], conf_v)
    pltpu.sync_copy(wealth_hbm, wealth_v)

    iota = lax.broadcasted_iota(jnp.int32, (LANES,), 0)
    ninf = jnp.full((LANES,), -jnp.inf, jnp.float32)

    def group(g, boff):
        tok = g * LANES + iota

        def combine(a, b):
            (va, ia), (vb, ib) = a, b
            p = vb > va
            return jnp.where(p, vb, va), jnp.where(p, ib, ia)

        def tourney(entries):
            while len(entries) > 1:
                entries = [combine(entries[i], entries[i + 1])
                           for i in range(0, len(entries), 2)]
            return entries[0]

        bw = []
        for blk in range(0, NUM_E, 8):
            leaves = []
            for e in range(blk, blk + 8):
                col = plsc.load_gather(conf_v, [tok * NUM_E + e])
                b = col * wealth_v[pl.ds(e * LANES, LANES)]
                bids_v[pl.ds(boff + e * LANES, LANES)] = b
                leaves.append((b, jnp.full((LANES,), e, jnp.int32)))
            bw.append(tourney(leaves))

        ms = []
        idxs = []
        for j in range(TOPK):
            m, midx = tourney(list(bw))
            ms.append(m)
            idxs.append(midx)
            if j == TOPK - 1:
                break
            plsc.store_scatter(bids_v, [boff + midx * LANES + iota], ninf)
            blkid = jnp.right_shift(midx, 3)
            blkbase = jnp.left_shift(blkid, 3)
            cand = []
            for s in range(8):
                eidx = blkbase + s
                v = plsc.load_gather(bids_v, [boff + jnp.left_shift(eidx, 4) + iota])
                cand.append((v, eidx))
            nv, ni = tourney(cand)
            for b in range(8):
                pb = blkid == b
                bw[b] = (jnp.where(pb, nv, bw[b][0]),
                         jnp.where(pb, ni, bw[b][1]))

        m0 = ms[0]
        zacc = [jnp.zeros((LANES,), jnp.float32) for _ in range(4)]
        for e in range(NUM_E):
            zacc[e % 4] = zacc[e % 4] + jnp.exp(
                bids_v[pl.ds(boff + e * LANES, LANES)] - m0)
        z = (zacc[0] + zacc[1]) + (zacc[2] + zacc[3])
        es = [jnp.exp(mj - m0) for mj in ms]
        z = z + ((es[0] + es[1]) + (es[2] + es[3])) + (
            (es[4] + es[5]) + (es[6] + es[7]))
        s = [ej / z for ej in es]
        denom = _fold_sum(s) + 1e-8
        rws = [sj / denom for sj in s]

        s8 = ms[0]
        for mj in ms[1:]:
            s8 = s8 + mj
        pays = []
        for j in range(TOPK):
            others = ms[:j] + ms[j + 1:]
            s7 = _fold_sum(others)
            pays.append(jnp.maximum(s7 - (s8 - ms[j]), 0.0))

        out_base = tok * TOPK
        for j in range(TOPK):
            plsc.store_scatter(sel_v, [out_base + j], idxs[j])
            plsc.store_scatter(rw_v, [out_base + j], rws[j])
            plsc.store_scatter(pay_v, [out_base + j], pays[j])

    def pair(i, _):
        group(2 * i, 0)
        group(2 * i + 1, NUM_E * LANES)
        return ()

    lax.fori_loop(0, tpw // LANES // 2, pair, (), unroll=False)

    pltpu.sync_copy(sel_v, sel_hbm.at[pl.ds(base * TOPK, tpw * TOPK)])
    pltpu.sync_copy(rw_v, rw_hbm.at[pl.ds(base * TOPK, tpw * TOPK)])
    pltpu.sync_copy(pay_v, pay_hbm.at[pl.ds(base * TOPK, tpw * TOPK)])


@jax.jit
def _run(confidences, wealth):
    b, s, e = confidences.shape
    t = b * s
    tpw = t // NW
    conf_flat = confidences.reshape(t * e)
    wealth_rep = jnp.broadcast_to(wealth[:, None], (e, LANES)).reshape(-1)
    mesh = plsc.VectorSubcoreMesh(core_axis_name="c", subcore_axis_name="s")
    wk = pl.kernel(
        _sc_body,
        out_type=[
            jax.ShapeDtypeStruct((t * TOPK,), jnp.int32),
            jax.ShapeDtypeStruct((t * TOPK,), jnp.float32),
            jax.ShapeDtypeStruct((t * TOPK,), jnp.float32),
        ],
        mesh=mesh,
        compiler_params=pltpu.CompilerParams(needs_layout_passes=False),
        scratch_types=[
            pltpu.VMEM((tpw * e,), jnp.float32),
            pltpu.VMEM((e * LANES,), jnp.float32),
            pltpu.VMEM((2 * NUM_E * LANES,), jnp.float32),
            pltpu.VMEM((tpw * TOPK,), jnp.int32),
            pltpu.VMEM((tpw * TOPK,), jnp.float32),
            pltpu.VMEM((tpw * TOPK,), jnp.float32),
        ],
    )
    sel, rw, pay = wk(conf_flat, wealth_rep)
    return (sel.reshape(b, s, TOPK), rw.reshape(b, s, TOPK),
            pay.reshape(b, s, TOPK))


def kernel(confidences, wealth):
    return _run(confidences, wealth)


# speedup vs baseline: 1.1613x; 1.1613x over previous
import jax
import jax.numpy as jnp
from jax import lax
from jax.experimental import pallas as pl
from jax.experimental.pallas import tpu as pltpu
from jax.experimental.pallas import tpu_sc as plsc

NUM_E = 64
TOPK = 8
LANES = 16
NC = 2
NS = 16
NW = NC * NS


def _fold_sum(vals):
    vals = list(vals)
    p = 1
    while p < len(vals):
        p *= 2
    while p > 1:
        p //= 2
        nxt = []
        for i in range(p):
            if i + p < len(vals):
                nxt.append(vals[i] + vals[i + p])
            elif i < len(vals):
                nxt.append(vals[i])
        vals = nxt
    return vals[0]


def _sc_body(conf_hbm, wealth_hbm, sel_hbm, rw_hbm, pay_hbm,
             conf_v, wealth_v, bids_v, sel_v, rw_v, pay_v):
    tpw = conf_v.shape[0]
    wid = lax.axis_index("s") * NC + lax.axis_index("c")
    base = wid * tpw
    pltpu.sync_copy(conf_hbm.at[pl.ds(base, tpw)], conf_v)
    pltpu.sync_copy(wealth_hbm, wealth_v)

    iota = lax.broadcasted_iota(jnp.int32, (LANES,), 0)
    ninf = jnp.full((LANES,), -jnp.inf, jnp.float32)

    def group(g, _):
        tok = g * LANES + iota

        def combine(a, b):
            (va, ia), (vb, ib) = a, b
            p = vb > va
            return jnp.where(p, vb, va), jnp.where(p, ib, ia)

        def tourney(entries):
            while len(entries) > 1:
                entries = [combine(entries[i], entries[i + 1])
                           for i in range(0, len(entries), 2)]
            return entries[0]

        bw = []
        for blk in range(0, NUM_E, 8):
            leaves = []
            for e in range(blk, blk + 8):
                col = plsc.load_gather(conf_v, [tok, jnp.full((LANES,), e, jnp.int32)])
                b = col * wealth_v[pl.ds(e * LANES, LANES)]
                bids_v[pl.ds(e * LANES, LANES)] = b
                leaves.append((b, jnp.full((LANES,), e, jnp.int32)))
            bw.append(tourney(leaves))

        ms = []
        idxs = []
        for j in range(TOPK):
            m, midx = tourney(list(bw))
            ms.append(m)
            idxs.append(midx)
            if j == TOPK - 1:
                break
            plsc.store_scatter(bids_v, [midx * LANES + iota], ninf)
            blkid = jnp.right_shift(midx, 3)
            blkbase = jnp.left_shift(blkid, 3)
            cand = []
            for s in range(8):
                eidx = blkbase + s
                v = plsc.load_gather(bids_v, [jnp.left_shift(eidx, 4) + iota])
                cand.append((v, eidx))
            nv, ni = tourney(cand)
            for b in range(8):
                pb = blkid == b
                bw[b] = (jnp.where(pb, nv, bw[b][0]),
                         jnp.where(pb, ni, bw[b][1]))

        m0 = ms[0]
        zacc = [jnp.zeros((LANES,), jnp.float32) for _ in range(4)]
        for e in range(NUM_E):
            zacc[e % 4] = zacc[e % 4] + jnp.exp(
                bids_v[pl.ds(e * LANES, LANES)] - m0)
        z = (zacc[0] + zacc[1]) + (zacc[2] + zacc[3])
        es = [jnp.exp(mj - m0) for mj in ms]
        z = z + ((es[0] + es[1]) + (es[2] + es[3])) + (
            (es[4] + es[5]) + (es[6] + es[7]))
        s = [ej / z for ej in es]
        denom = _fold_sum(s) + 1e-8
        rws = [sj / denom for sj in s]

        s8 = ms[0]
        for mj in ms[1:]:
            s8 = s8 + mj
        pays = []
        for j in range(TOPK):
            others = ms[:j] + ms[j + 1:]
            s7 = _fold_sum(others)
            pays.append(jnp.maximum(s7 - (s8 - ms[j]), 0.0))

        for j in range(TOPK):
            jcol = jnp.full((LANES,), j, jnp.int32)
            plsc.store_scatter(sel_v, [tok, jcol], idxs[j])
            plsc.store_scatter(rw_v, [tok, jcol], rws[j])
            plsc.store_scatter(pay_v, [tok, jcol], pays[j])
        return ()

    lax.fori_loop(0, tpw // LANES, group, (), unroll=False)

    pltpu.sync_copy(sel_v, sel_hbm.at[pl.ds(base, tpw)])
    pltpu.sync_copy(rw_v, rw_hbm.at[pl.ds(base, tpw)])
    pltpu.sync_copy(pay_v, pay_hbm.at[pl.ds(base, tpw)])


@jax.jit
def _run(confidences, wealth):
    b, s, e = confidences.shape
    t = b * s
    tpw = t // NW
    conf2 = confidences.reshape(t, e)
    wealth_rep = jnp.broadcast_to(wealth[:, None], (e, LANES)).reshape(-1)
    mesh = plsc.VectorSubcoreMesh(core_axis_name="c", subcore_axis_name="s")
    wk = pl.kernel(
        _sc_body,
        out_type=[
            jax.ShapeDtypeStruct((t, TOPK), jnp.int32),
            jax.ShapeDtypeStruct((t, TOPK), jnp.float32),
            jax.ShapeDtypeStruct((t, TOPK), jnp.float32),
        ],
        mesh=mesh,
        compiler_params=pltpu.CompilerParams(needs_layout_passes=False, use_tc_tiling_on_sc=False),
        scratch_types=[
            pltpu.VMEM((tpw, e), jnp.float32),
            pltpu.VMEM((e * LANES,), jnp.float32),
            pltpu.VMEM((NUM_E * LANES,), jnp.float32),
            pltpu.VMEM((tpw, TOPK), jnp.int32),
            pltpu.VMEM((tpw, TOPK), jnp.float32),
            pltpu.VMEM((tpw, TOPK), jnp.float32),
        ],
    )
    sel, rw, pay = wk(conf2, wealth_rep)
    return (sel.reshape(b, s, TOPK), rw.reshape(b, s, TOPK),
            pay.reshape(b, s, TOPK))


def kernel(confidences, wealth):
    return _run(confidences, wealth)
